# baseline (speedup 1.0000x reference)
import functools

import jax
import jax.numpy as jnp
from jax import lax
from jax.experimental import pallas as pl
from jax.experimental.pallas import tpu as pltpu
from jax.experimental.pallas import tpu_sc as plsc

N = 10000
D = 128
E = 320000

NC = 2
NS = 16
NW = NC * NS
EPW = E // NW
CHUNK = 100
NCHUNK = EPW // CHUNK
NPAIR = NCHUNK // 2
N_PAD = 10240
RPT = N_PAD // NS

_MESH = plsc.VectorSubcoreMesh(
    core_axis_name="c", subcore_axis_name="s", num_cores=NC, num_subcores=NS
)


def _deg_body(ei_hbm, ones_hbm, zeros_hbm, out_hbm, deg_sh, dstv, onesv):
    c = lax.axis_index("c")
    s = lax.axis_index("s")
    w = s * NC + c
    row0 = s * RPT
    pltpu.sync_copy(zeros_hbm.at[pl.ds(row0, RPT)], deg_sh.at[pl.ds(row0, RPT)])
    pltpu.sync_copy(ones_hbm, onesv)
    pltpu.sync_copy(ei_hbm.at[1, w], dstv)
    plsc.subcore_barrier()

    def body(i, carry):
        pltpu.sync_copy(onesv, deg_sh.at[dstv.at[i]], add=True)
        return carry

    lax.fori_loop(0, NCHUNK, body, 0)
    plsc.subcore_barrier()
    pltpu.sync_copy(deg_sh.at[pl.ds(row0, RPT)], out_hbm.at[c, pl.ds(row0, RPT)])


_deg_call = functools.partial(
    pl.kernel,
    out_type=jax.ShapeDtypeStruct((NC, N_PAD, 16), jnp.float32),
    mesh=_MESH,
    compiler_params=pltpu.CompilerParams(use_tc_tiling_on_sc=False),
    scratch_types=[
        pltpu.VMEM_SHARED((N_PAD, 16), jnp.float32),
        pltpu.VMEM((NCHUNK, CHUNK), jnp.int32),
        pltpu.VMEM((CHUNK, 16), jnp.float32),
    ],
)(_deg_body)


def _hop_body(h_hbm, ei_hbm, zeros_hbm, out_hbm,
              acc_sh, srcv, dstv, rb0, rb1, semg0, semg1, sems0, sems1):
    c = lax.axis_index("c")
    s = lax.axis_index("s")
    w = s * NC + c
    row0 = s * RPT
    pltpu.sync_copy(zeros_hbm.at[pl.ds(row0, RPT)], acc_sh.at[pl.ds(row0, RPT)])
    pltpu.sync_copy(ei_hbm.at[0, w], srcv)
    pltpu.sync_copy(ei_hbm.at[1, w], dstv)
    plsc.subcore_barrier()

    pltpu.make_async_copy(h_hbm.at[srcv.at[0]], rb0, semg0).start()

    def body(j, carry):
        a = 2 * j
        b = a + 1
        pltpu.make_async_copy(h_hbm.at[srcv.at[b]], rb1, semg1).start()
        pltpu.make_async_copy(h_hbm.at[srcv.at[a]], rb0, semg0).wait()
        pltpu.sync_copy(rb0, acc_sh.at[dstv.at[a]], add=True)

        @pl.when(a + 2 < NCHUNK)
        def _():
            pltpu.make_async_copy(h_hbm.at[srcv.at[a + 2]], rb0, semg0).start()

        pltpu.make_async_copy(h_hbm.at[srcv.at[b]], rb1, semg1).wait()
        pltpu.sync_copy(rb1, acc_sh.at[dstv.at[b]], add=True)
        return carry

    lax.fori_loop(0, NPAIR, body, 0)
    if NCHUNK % 2:
        last = NCHUNK - 1
        pltpu.make_async_copy(h_hbm.at[srcv.at[last]], rb0, semg0).wait()
        pltpu.sync_copy(rb0, acc_sh.at[dstv.at[last]], add=True)

    plsc.subcore_barrier()
    pltpu.sync_copy(acc_sh.at[pl.ds(row0, RPT)], out_hbm.at[c, pl.ds(row0, RPT)])


_hop_call = functools.partial(
    pl.kernel,
    out_type=jax.ShapeDtypeStruct((NC, N_PAD, D), jnp.float32),
    mesh=_MESH,
    compiler_params=pltpu.CompilerParams(use_tc_tiling_on_sc=False),
    scratch_types=[
        pltpu.VMEM_SHARED((N_PAD, D), jnp.float32),
        pltpu.VMEM((NCHUNK, CHUNK), jnp.int32),
        pltpu.VMEM((NCHUNK, CHUNK), jnp.int32),
        pltpu.VMEM((CHUNK, D), jnp.float32),
        pltpu.VMEM((CHUNK, D), jnp.float32),
        pltpu.SemaphoreType.DMA,
        pltpu.SemaphoreType.DMA,
        pltpu.SemaphoreType.DMA,
        pltpu.SemaphoreType.DMA,
    ],
)(_hop_body)


BLK = 1000
GRID = N // BLK
_SQRT1_2 = 0.7071067811865476


def _gelu(h):
    return 0.5 * h * (1.0 + lax.erf(h * _SQRT1_2))


def _layernorm(h, g, b):
    mu = jnp.mean(h, axis=1, keepdims=True)
    var = jnp.mean((h - mu) * (h - mu), axis=1, keepdims=True)
    return (h - mu) * lax.rsqrt(var + 1e-5) * g + b


def _matmul_t(a, w):
    return lax.dot_general(a, w, (((1,), (1,)), ((), ())),
                           preferred_element_type=jnp.float32)


def _pre_body(x_ref, w1_ref, b1_ref, g1_ref, bt1_ref, h_ref):
    h = _matmul_t(x_ref[...], w1_ref[...]) + b1_ref[...]
    h_ref[...] = _layernorm(_gelu(h), g1_ref[...], bt1_ref[...])


_pre_call = pl.pallas_call(
    _pre_body,
    grid=(GRID,),
    in_specs=[
        pl.BlockSpec((BLK, D), lambda i: (i, 0)),
        pl.BlockSpec((D, D), lambda i: (0, 0)),
        pl.BlockSpec((1, D), lambda i: (0, 0)),
        pl.BlockSpec((1, D), lambda i: (0, 0)),
        pl.BlockSpec((1, D), lambda i: (0, 0)),
    ],
    out_specs=pl.BlockSpec((BLK, D), lambda i: (i, 0)),
    out_shape=jax.ShapeDtypeStruct((N, D), jnp.float32),
)


def _scale_body(h_ref, degp_ref, h1_ref, dinv_ref):
    deg = degp_ref[0, :, 0:1] + degp_ref[1, :, 0:1] + 1.0
    dinv = lax.rsqrt(deg)
    dinv_ref[...] = dinv
    h1_ref[...] = h_ref[...] * jnp.broadcast_to(dinv, (BLK, D))


_scale_call = pl.pallas_call(
    _scale_body,
    grid=(GRID,),
    in_specs=[
        pl.BlockSpec((BLK, D), lambda i: (i, 0)),
        pl.BlockSpec((NC, BLK, 16), lambda i: (0, i, 0)),
    ],
    out_specs=[
        pl.BlockSpec((BLK, D), lambda i: (i, 0)),
        pl.BlockSpec((BLK, 1), lambda i: (i, 0)),
    ],
    out_shape=[
        jax.ShapeDtypeStruct((N, D), jnp.float32),
        jax.ShapeDtypeStruct((N, 1), jnp.float32),
    ],
)


def _mid_body(p_ref, h1_ref, dinv_ref, o_ref):
    s = p_ref[0] + p_ref[1] + h1_ref[...]
    dv = dinv_ref[...]
    o_ref[...] = jnp.broadcast_to(dv * dv, (BLK, D)) * s


_mid_call = pl.pallas_call(
    _mid_body,
    grid=(GRID,),
    in_specs=[
        pl.BlockSpec((NC, BLK, D), lambda i: (0, i, 0)),
        pl.BlockSpec((BLK, D), lambda i: (i, 0)),
        pl.BlockSpec((BLK, 1), lambda i: (i, 0)),
    ],
    out_specs=pl.BlockSpec((BLK, D), lambda i: (i, 0)),
    out_shape=jax.ShapeDtypeStruct((N, D), jnp.float32),
)


def _dense_out_body(q_ref, h2_ref, dinv_ref, wc_ref, bc_ref, g2_ref, bt2_ref,
                    w2_ref, b2_ref, o_ref):
    y = jnp.broadcast_to(dinv_ref[...], (BLK, D)) * (
        q_ref[0] + q_ref[1] + h2_ref[...])
    h = _matmul_t(y, wc_ref[...]) + bc_ref[...]
    h = _layernorm(_gelu(h), g2_ref[...], bt2_ref[...])
    o_ref[...] = _matmul_t(h, w2_ref[...]) + b2_ref[...]


_dense_out_call = pl.pallas_call(
    _dense_out_body,
    grid=(GRID,),
    in_specs=[
        pl.BlockSpec((NC, BLK, D), lambda i: (0, i, 0)),
        pl.BlockSpec((BLK, D), lambda i: (i, 0)),
        pl.BlockSpec((BLK, 1), lambda i: (i, 0)),
        pl.BlockSpec((D, D), lambda i: (0, 0)),
        pl.BlockSpec((1, D), lambda i: (0, 0)),
        pl.BlockSpec((1, D), lambda i: (0, 0)),
        pl.BlockSpec((1, D), lambda i: (0, 0)),
        pl.BlockSpec((D, D), lambda i: (0, 0)),
        pl.BlockSpec((1, D), lambda i: (0, 0)),
    ],
    out_specs=pl.BlockSpec((BLK, D), lambda i: (i, 0)),
    out_shape=jax.ShapeDtypeStruct((N, D), jnp.float32),
)


def kernel(x, edge_index, W1, b1, g1, bt1, Wc, bc, g2, bt2, W2, b2):
    ei4 = edge_index.reshape(2, NW, NCHUNK, CHUNK)
    ones16 = jnp.ones((CHUNK, 16), jnp.float32)
    zeros16 = jnp.zeros((N_PAD, 16), jnp.float32)
    zerosD = jnp.zeros((N_PAD, D), jnp.float32)

    degp = _deg_call(ei4, ones16, zeros16)
    h = _pre_call(x, W1, b1.reshape(1, D), g1.reshape(1, D), bt1.reshape(1, D))
    h1, dinv = _scale_call(h, degp)
    p = _hop_call(h1, ei4, zerosD)
    h2 = _mid_call(p, h1, dinv)
    q = _hop_call(h2, ei4, zerosD)
    return _dense_out_call(q, h2, dinv, Wc, bc.reshape(1, D),
                           g2.reshape(1, D), bt2.reshape(1, D), W2,
                           b2.reshape(1, D))


# speedup vs baseline: 1.0365x; 1.0365x over previous
import functools

import jax
import jax.numpy as jnp
from jax import lax
from jax.experimental import pallas as pl
from jax.experimental.pallas import tpu as pltpu
from jax.experimental.pallas import tpu_sc as plsc

N = 10000
D = 128
E = 320000

NC = 2
NS = 16
NW = NC * NS
EPW = E // NW
CHUNK = 100
NCHUNK = EPW // CHUNK
NPAIR = NCHUNK // 2
N_PAD = 10240
RPT = N_PAD // NS

_MESH = plsc.VectorSubcoreMesh(
    core_axis_name="c", subcore_axis_name="s", num_cores=NC, num_subcores=NS
)


def _deg_body(ei_hbm, ones_hbm, zeros_hbm, out_hbm, deg_sh, dstv, onesv, semd):
    c = lax.axis_index("c")
    s = lax.axis_index("s")
    w = s * NC + c
    row0 = s * RPT
    pltpu.sync_copy(zeros_hbm.at[pl.ds(row0, RPT)], deg_sh.at[pl.ds(row0, RPT)])
    pltpu.sync_copy(ones_hbm, onesv)
    pltpu.sync_copy(ei_hbm.at[1, w], dstv)
    plsc.subcore_barrier()

    def body(i, carry):
        pltpu.make_async_copy(onesv, deg_sh.at[dstv.at[i]], semd).start(add=True)
        return carry

    lax.fori_loop(0, NCHUNK, body, 0)

    def drain(i, carry):
        pltpu.make_async_copy(onesv, deg_sh.at[dstv.at[0]], semd).wait()
        return carry

    lax.fori_loop(0, NCHUNK, drain, 0)
    plsc.subcore_barrier()
    pltpu.sync_copy(deg_sh.at[pl.ds(row0, RPT)], out_hbm.at[c, pl.ds(row0, RPT)])


_deg_call = functools.partial(
    pl.kernel,
    out_type=jax.ShapeDtypeStruct((NC, N_PAD, 16), jnp.float32),
    mesh=_MESH,
    compiler_params=pltpu.CompilerParams(use_tc_tiling_on_sc=False),
    scratch_types=[
        pltpu.VMEM_SHARED((N_PAD, 16), jnp.float32),
        pltpu.VMEM((NCHUNK, CHUNK), jnp.int32),
        pltpu.VMEM((CHUNK, 16), jnp.float32),
        pltpu.SemaphoreType.DMA,
    ],
)(_deg_body)


def _hop_body(h_hbm, ei_hbm, zeros_hbm, out_hbm,
              acc_sh, srcv, dstv, rb0, rb1, semg0, semg1, sems0, sems1):
    c = lax.axis_index("c")
    s = lax.axis_index("s")
    w = s * NC + c
    row0 = s * RPT
    pltpu.sync_copy(zeros_hbm.at[pl.ds(row0, RPT)], acc_sh.at[pl.ds(row0, RPT)])
    pltpu.sync_copy(ei_hbm.at[0, w], srcv)
    pltpu.sync_copy(ei_hbm.at[1, w], dstv)
    plsc.subcore_barrier()

    pltpu.make_async_copy(h_hbm.at[srcv.at[0]], rb0, semg0).start()

    def body(j, carry):
        a = 2 * j
        b = a + 1
        pltpu.make_async_copy(h_hbm.at[srcv.at[b]], rb1, semg1).start()
        pltpu.make_async_copy(h_hbm.at[srcv.at[a]], rb0, semg0).wait()
        pltpu.sync_copy(rb0, acc_sh.at[dstv.at[a]], add=True)

        @pl.when(a + 2 < NCHUNK)
        def _():
            pltpu.make_async_copy(h_hbm.at[srcv.at[a + 2]], rb0, semg0).start()

        pltpu.make_async_copy(h_hbm.at[srcv.at[b]], rb1, semg1).wait()
        pltpu.sync_copy(rb1, acc_sh.at[dstv.at[b]], add=True)
        return carry

    lax.fori_loop(0, NPAIR, body, 0)
    if NCHUNK % 2:
        last = NCHUNK - 1
        pltpu.make_async_copy(h_hbm.at[srcv.at[last]], rb0, semg0).wait()
        pltpu.sync_copy(rb0, acc_sh.at[dstv.at[last]], add=True)

    plsc.subcore_barrier()
    pltpu.sync_copy(acc_sh.at[pl.ds(row0, RPT)], out_hbm.at[c, pl.ds(row0, RPT)])


_hop_call = functools.partial(
    pl.kernel,
    out_type=jax.ShapeDtypeStruct((NC, N_PAD, D), jnp.float32),
    mesh=_MESH,
    compiler_params=pltpu.CompilerParams(use_tc_tiling_on_sc=False),
    scratch_types=[
        pltpu.VMEM_SHARED((N_PAD, D), jnp.float32),
        pltpu.VMEM((NCHUNK, CHUNK), jnp.int32),
        pltpu.VMEM((NCHUNK, CHUNK), jnp.int32),
        pltpu.VMEM((CHUNK, D), jnp.float32),
        pltpu.VMEM((CHUNK, D), jnp.float32),
        pltpu.SemaphoreType.DMA,
        pltpu.SemaphoreType.DMA,
        pltpu.SemaphoreType.DMA,
        pltpu.SemaphoreType.DMA,
    ],
)(_hop_body)


BLK = 2000
GRID = N // BLK
_SQRT1_2 = 0.7071067811865476


def _gelu(h):
    return 0.5 * h * (1.0 + lax.erf(h * _SQRT1_2))


def _layernorm(h, g, b):
    mu = jnp.mean(h, axis=1, keepdims=True)
    var = jnp.mean((h - mu) * (h - mu), axis=1, keepdims=True)
    return (h - mu) * lax.rsqrt(var + 1e-5) * g + b


def _matmul_t(a, w):
    return lax.dot_general(a, w, (((1,), (1,)), ((), ())),
                           preferred_element_type=jnp.float32)


def _pre_body(x_ref, w1_ref, b1_ref, g1_ref, bt1_ref, h_ref):
    h = _matmul_t(x_ref[...], w1_ref[...]) + b1_ref[...]
    h_ref[...] = _layernorm(_gelu(h), g1_ref[...], bt1_ref[...])


_pre_call = pl.pallas_call(
    _pre_body,
    grid=(GRID,),
    in_specs=[
        pl.BlockSpec((BLK, D), lambda i: (i, 0)),
        pl.BlockSpec((D, D), lambda i: (0, 0)),
        pl.BlockSpec((1, D), lambda i: (0, 0)),
        pl.BlockSpec((1, D), lambda i: (0, 0)),
        pl.BlockSpec((1, D), lambda i: (0, 0)),
    ],
    out_specs=pl.BlockSpec((BLK, D), lambda i: (i, 0)),
    out_shape=jax.ShapeDtypeStruct((N, D), jnp.float32),
)


def _scale_body(h_ref, degp_ref, h1_ref, dinv_ref):
    deg = degp_ref[0, :, 0:1] + degp_ref[1, :, 0:1] + 1.0
    dinv = lax.rsqrt(deg)
    dinv_ref[...] = dinv
    h1_ref[...] = h_ref[...] * jnp.broadcast_to(dinv, (BLK, D))


_scale_call = pl.pallas_call(
    _scale_body,
    grid=(GRID,),
    in_specs=[
        pl.BlockSpec((BLK, D), lambda i: (i, 0)),
        pl.BlockSpec((NC, BLK, 16), lambda i: (0, i, 0)),
    ],
    out_specs=[
        pl.BlockSpec((BLK, D), lambda i: (i, 0)),
        pl.BlockSpec((BLK, 1), lambda i: (i, 0)),
    ],
    out_shape=[
        jax.ShapeDtypeStruct((N, D), jnp.float32),
        jax.ShapeDtypeStruct((N, 1), jnp.float32),
    ],
)


def _mid_body(p_ref, h1_ref, dinv_ref, o_ref):
    s = p_ref[0] + p_ref[1] + h1_ref[...]
    dv = dinv_ref[...]
    o_ref[...] = jnp.broadcast_to(dv * dv, (BLK, D)) * s


_mid_call = pl.pallas_call(
    _mid_body,
    grid=(GRID,),
    in_specs=[
        pl.BlockSpec((NC, BLK, D), lambda i: (0, i, 0)),
        pl.BlockSpec((BLK, D), lambda i: (i, 0)),
        pl.BlockSpec((BLK, 1), lambda i: (i, 0)),
    ],
    out_specs=pl.BlockSpec((BLK, D), lambda i: (i, 0)),
    out_shape=jax.ShapeDtypeStruct((N, D), jnp.float32),
)


def _dense_out_body(q_ref, h2_ref, dinv_ref, wc_ref, bc_ref, g2_ref, bt2_ref,
                    w2_ref, b2_ref, o_ref):
    y = jnp.broadcast_to(dinv_ref[...], (BLK, D)) * (
        q_ref[0] + q_ref[1] + h2_ref[...])
    h = _matmul_t(y, wc_ref[...]) + bc_ref[...]
    h = _layernorm(_gelu(h), g2_ref[...], bt2_ref[...])
    o_ref[...] = _matmul_t(h, w2_ref[...]) + b2_ref[...]


_dense_out_call = pl.pallas_call(
    _dense_out_body,
    grid=(GRID,),
    in_specs=[
        pl.BlockSpec((NC, BLK, D), lambda i: (0, i, 0)),
        pl.BlockSpec((BLK, D), lambda i: (i, 0)),
        pl.BlockSpec((BLK, 1), lambda i: (i, 0)),
        pl.BlockSpec((D, D), lambda i: (0, 0)),
        pl.BlockSpec((1, D), lambda i: (0, 0)),
        pl.BlockSpec((1, D), lambda i: (0, 0)),
        pl.BlockSpec((1, D), lambda i: (0, 0)),
        pl.BlockSpec((D, D), lambda i: (0, 0)),
        pl.BlockSpec((1, D), lambda i: (0, 0)),
    ],
    out_specs=pl.BlockSpec((BLK, D), lambda i: (i, 0)),
    out_shape=jax.ShapeDtypeStruct((N, D), jnp.float32),
)


def kernel(x, edge_index, W1, b1, g1, bt1, Wc, bc, g2, bt2, W2, b2):
    ei4 = edge_index.reshape(2, NW, NCHUNK, CHUNK)
    ones16 = jnp.ones((CHUNK, 16), jnp.float32)
    zeros16 = jnp.zeros((N_PAD, 16), jnp.float32)
    zerosD = jnp.zeros((N_PAD, D), jnp.float32)

    degp = _deg_call(ei4, ones16, zeros16)
    h = _pre_call(x, W1, b1.reshape(1, D), g1.reshape(1, D), bt1.reshape(1, D))
    h1, dinv = _scale_call(h, degp)
    p = _hop_call(h1, ei4, zerosD)
    h2 = _mid_call(p, h1, dinv)
    q = _hop_call(h2, ei4, zerosD)
    return _dense_out_call(q, h2, dinv, Wc, bc.reshape(1, D),
                           g2.reshape(1, D), bt2.reshape(1, D), W2,
                           b2.reshape(1, D))


# speedup vs baseline: 1.0463x; 1.0094x over previous
import functools

import jax
import jax.numpy as jnp
from jax import lax
from jax.experimental import pallas as pl
from jax.experimental.pallas import tpu as pltpu
from jax.experimental.pallas import tpu_sc as plsc

N = 10000
D = 128
E = 320000

NC = 2
NS = 16
NW = NC * NS
EPW = E // NW
CHUNK = 100
NCHUNK = EPW // CHUNK
NPAIR = NCHUNK // 2
N_PAD = 10240
RPT = N_PAD // NS

_MESH = plsc.VectorSubcoreMesh(
    core_axis_name="c", subcore_axis_name="s", num_cores=NC, num_subcores=NS
)


def _deg_body(ei_hbm, ones_hbm, zeros_hbm, out_hbm, deg_sh, dstv, onesv, semd):
    c = lax.axis_index("c")
    s = lax.axis_index("s")
    w = s * NC + c
    row0 = s * RPT
    pltpu.sync_copy(zeros_hbm.at[pl.ds(row0, RPT)], deg_sh.at[pl.ds(row0, RPT)])
    pltpu.sync_copy(ones_hbm, onesv)
    pltpu.sync_copy(ei_hbm.at[1, w], dstv)
    plsc.subcore_barrier()

    def body(i, carry):
        pltpu.make_async_copy(onesv, deg_sh.at[dstv.at[i]], semd).start(add=True)
        return carry

    lax.fori_loop(0, NCHUNK, body, 0)

    def drain(i, carry):
        pltpu.make_async_copy(onesv, deg_sh.at[dstv.at[0]], semd).wait()
        return carry

    lax.fori_loop(0, NCHUNK, drain, 0)
    plsc.subcore_barrier()
    pltpu.sync_copy(deg_sh.at[pl.ds(row0, RPT)], out_hbm.at[c, pl.ds(row0, RPT)])


_deg_call = functools.partial(
    pl.kernel,
    out_type=jax.ShapeDtypeStruct((NC, N_PAD, 16), jnp.float32),
    mesh=_MESH,
    compiler_params=pltpu.CompilerParams(use_tc_tiling_on_sc=False),
    scratch_types=[
        pltpu.VMEM_SHARED((N_PAD, 16), jnp.float32),
        pltpu.VMEM((NCHUNK, CHUNK), jnp.int32),
        pltpu.VMEM((CHUNK, 16), jnp.float32),
        pltpu.SemaphoreType.DMA,
    ],
)(_deg_body)


def _hop_body(h_hbm, ei_hbm, zeros_hbm, out_hbm,
              acc_sh, srcv, dstv, rb0, rb1, semg0, semg1, sems0, sems1):
    c = lax.axis_index("c")
    s = lax.axis_index("s")
    w = s * NC + c
    row0 = s * RPT
    pltpu.sync_copy(ei_hbm.at[0, w], srcv)
    pltpu.sync_copy(ei_hbm.at[1, w], dstv)
    pltpu.make_async_copy(h_hbm.at[srcv.at[0]], rb0, semg0).start()
    pltpu.make_async_copy(h_hbm.at[srcv.at[1]], rb1, semg1).start()
    pltpu.sync_copy(zeros_hbm.at[pl.ds(row0, RPT)], acc_sh.at[pl.ds(row0, RPT)])
    plsc.subcore_barrier()

    def body(j, carry):
        a = 2 * j
        b = a + 1
        pltpu.make_async_copy(h_hbm.at[srcv.at[a]], rb0, semg0).wait()
        pltpu.sync_copy(rb0, acc_sh.at[dstv.at[a]], add=True)

        @pl.when(a + 2 < NCHUNK)
        def _():
            pltpu.make_async_copy(h_hbm.at[srcv.at[a + 2]], rb0, semg0).start()

        pltpu.make_async_copy(h_hbm.at[srcv.at[b]], rb1, semg1).wait()
        pltpu.sync_copy(rb1, acc_sh.at[dstv.at[b]], add=True)

        @pl.when(b + 2 < NCHUNK)
        def _():
            pltpu.make_async_copy(h_hbm.at[srcv.at[b + 2]], rb1, semg1).start()

        return carry

    lax.fori_loop(0, NPAIR, body, 0)
    if NCHUNK % 2:
        last = NCHUNK - 1
        pltpu.make_async_copy(h_hbm.at[srcv.at[last]], rb0, semg0).wait()
        pltpu.sync_copy(rb0, acc_sh.at[dstv.at[last]], add=True)

    plsc.subcore_barrier()
    pltpu.sync_copy(acc_sh.at[pl.ds(row0, RPT)], out_hbm.at[c, pl.ds(row0, RPT)])


_hop_call = functools.partial(
    pl.kernel,
    out_type=jax.ShapeDtypeStruct((NC, N_PAD, D), jnp.float32),
    mesh=_MESH,
    compiler_params=pltpu.CompilerParams(use_tc_tiling_on_sc=False),
    scratch_types=[
        pltpu.VMEM_SHARED((N_PAD, D), jnp.float32),
        pltpu.VMEM((NCHUNK, CHUNK), jnp.int32),
        pltpu.VMEM((NCHUNK, CHUNK), jnp.int32),
        pltpu.VMEM((CHUNK, D), jnp.float32),
        pltpu.VMEM((CHUNK, D), jnp.float32),
        pltpu.SemaphoreType.DMA,
        pltpu.SemaphoreType.DMA,
        pltpu.SemaphoreType.DMA,
        pltpu.SemaphoreType.DMA,
    ],
)(_hop_body)


BLK = 2000
GRID = N // BLK
_SQRT1_2 = 0.7071067811865476


def _gelu(h):
    return 0.5 * h * (1.0 + lax.erf(h * _SQRT1_2))


def _layernorm(h, g, b):
    mu = jnp.mean(h, axis=1, keepdims=True)
    var = jnp.mean((h - mu) * (h - mu), axis=1, keepdims=True)
    return (h - mu) * lax.rsqrt(var + 1e-5) * g + b


def _matmul_t(a, w):
    return lax.dot_general(a, w, (((1,), (1,)), ((), ())),
                           preferred_element_type=jnp.float32)


def _pre_body(x_ref, w1_ref, b1_ref, g1_ref, bt1_ref, h_ref):
    h = _matmul_t(x_ref[...], w1_ref[...]) + b1_ref[...]
    h_ref[...] = _layernorm(_gelu(h), g1_ref[...], bt1_ref[...])


_pre_call = pl.pallas_call(
    _pre_body,
    grid=(GRID,),
    in_specs=[
        pl.BlockSpec((BLK, D), lambda i: (i, 0)),
        pl.BlockSpec((D, D), lambda i: (0, 0)),
        pl.BlockSpec((1, D), lambda i: (0, 0)),
        pl.BlockSpec((1, D), lambda i: (0, 0)),
        pl.BlockSpec((1, D), lambda i: (0, 0)),
    ],
    out_specs=pl.BlockSpec((BLK, D), lambda i: (i, 0)),
    out_shape=jax.ShapeDtypeStruct((N, D), jnp.float32),
)


def _scale_body(h_ref, degp_ref, h1_ref, dinv_ref):
    deg = degp_ref[0, :, 0:1] + degp_ref[1, :, 0:1] + 1.0
    dinv = lax.rsqrt(deg)
    dinv_ref[...] = dinv
    h1_ref[...] = h_ref[...] * jnp.broadcast_to(dinv, (BLK, D))


_scale_call = pl.pallas_call(
    _scale_body,
    grid=(GRID,),
    in_specs=[
        pl.BlockSpec((BLK, D), lambda i: (i, 0)),
        pl.BlockSpec((NC, BLK, 16), lambda i: (0, i, 0)),
    ],
    out_specs=[
        pl.BlockSpec((BLK, D), lambda i: (i, 0)),
        pl.BlockSpec((BLK, 1), lambda i: (i, 0)),
    ],
    out_shape=[
        jax.ShapeDtypeStruct((N, D), jnp.float32),
        jax.ShapeDtypeStruct((N, 1), jnp.float32),
    ],
)


def _mid_body(p_ref, h1_ref, dinv_ref, o_ref):
    s = p_ref[0] + p_ref[1] + h1_ref[...]
    dv = dinv_ref[...]
    o_ref[...] = jnp.broadcast_to(dv * dv, (BLK, D)) * s


_mid_call = pl.pallas_call(
    _mid_body,
    grid=(GRID,),
    in_specs=[
        pl.BlockSpec((NC, BLK, D), lambda i: (0, i, 0)),
        pl.BlockSpec((BLK, D), lambda i: (i, 0)),
        pl.BlockSpec((BLK, 1), lambda i: (i, 0)),
    ],
    out_specs=pl.BlockSpec((BLK, D), lambda i: (i, 0)),
    out_shape=jax.ShapeDtypeStruct((N, D), jnp.float32),
)


def _dense_out_body(q_ref, h2_ref, dinv_ref, wc_ref, bc_ref, g2_ref, bt2_ref,
                    w2_ref, b2_ref, o_ref):
    y = jnp.broadcast_to(dinv_ref[...], (BLK, D)) * (
        q_ref[0] + q_ref[1] + h2_ref[...])
    h = _matmul_t(y, wc_ref[...]) + bc_ref[...]
    h = _layernorm(_gelu(h), g2_ref[...], bt2_ref[...])
    o_ref[...] = _matmul_t(h, w2_ref[...]) + b2_ref[...]


_dense_out_call = pl.pallas_call(
    _dense_out_body,
    grid=(GRID,),
    in_specs=[
        pl.BlockSpec((NC, BLK, D), lambda i: (0, i, 0)),
        pl.BlockSpec((BLK, D), lambda i: (i, 0)),
        pl.BlockSpec((BLK, 1), lambda i: (i, 0)),
        pl.BlockSpec((D, D), lambda i: (0, 0)),
        pl.BlockSpec((1, D), lambda i: (0, 0)),
        pl.BlockSpec((1, D), lambda i: (0, 0)),
        pl.BlockSpec((1, D), lambda i: (0, 0)),
        pl.BlockSpec((D, D), lambda i: (0, 0)),
        pl.BlockSpec((1, D), lambda i: (0, 0)),
    ],
    out_specs=pl.BlockSpec((BLK, D), lambda i: (i, 0)),
    out_shape=jax.ShapeDtypeStruct((N, D), jnp.float32),
)


def kernel(x, edge_index, W1, b1, g1, bt1, Wc, bc, g2, bt2, W2, b2):
    ei4 = edge_index.reshape(2, NW, NCHUNK, CHUNK)
    ones16 = jnp.ones((CHUNK, 16), jnp.float32)
    zeros16 = jnp.zeros((N_PAD, 16), jnp.float32)
    zerosD = jnp.zeros((N_PAD, D), jnp.float32)

    degp = _deg_call(ei4, ones16, zeros16)
    h = _pre_call(x, W1, b1.reshape(1, D), g1.reshape(1, D), bt1.reshape(1, D))
    h1, dinv = _scale_call(h, degp)
    p = _hop_call(h1, ei4, zerosD)
    h2 = _mid_call(p, h1, dinv)
    q = _hop_call(h2, ei4, zerosD)
    return _dense_out_call(q, h2, dinv, Wc, bc.reshape(1, D),
                           g2.reshape(1, D), bt2.reshape(1, D), W2,
                           b2.reshape(1, D))
